# baseline (speedup 1.0000x reference)
import functools

import jax
import jax.numpy as jnp
import numpy as np
from jax.experimental import pallas as pl
from jax.experimental.pallas import tpu as pltpu

FRAG = 32
G_PER_HEAD = 3
PROTO_COUNT = 16
TOP_U = 8
TOPK_FRAC = 0.2
W_MEAN, W_MAX, W_TOPK, W_STD = 1.0, 0.6, 0.4, 0.2

T_Q = 256
GPAD = 8


def _normalize_safe(x, eps=1e-6):
    n = jnp.linalg.norm(x, axis=-1, keepdims=True)
    return x / jnp.maximum(n, eps)


def _select_kernel(kbar_ref, qp_ref, o_ref, smat_ref, *, S, P, U):
    kb = kbar_ref[0]
    qp = qp_ref[0]
    smat = jax.nn.relu(jax.lax.dot_general(
        kb, qp, (((1,), (1,)), ((), ())),
        preferred_element_type=jnp.float32))
    smat_ref[...] = smat

    lane = jax.lax.broadcasted_iota(jnp.int32, (S, P), 1)
    mean = jnp.mean(smat, axis=-1, keepdims=True)
    mx = jnp.max(smat, axis=-1, keepdims=True)
    kq = max(1, int(round(P * TOPK_FRAC)))
    cur = smat
    s3 = jnp.zeros((S, 1), jnp.float32)
    for _ in range(kq):
        mi = jnp.max(cur, axis=-1, keepdims=True)
        first = jnp.min(jnp.where(cur == mi, lane, P), axis=-1, keepdims=True)
        s3 = s3 + mi
        cur = jnp.where(lane == first, -jnp.inf, cur)
    topk_mean = s3 / float(kq)
    dev = smat - mean
    std = jnp.sqrt(jnp.sum(dev * dev, axis=-1, keepdims=True) / (P - 1))
    u = W_MEAN * mean + W_MAX * mx + W_TOPK * topk_mean + W_STD * std

    row = jax.lax.broadcasted_iota(jnp.int32, (S, 1), 0)
    val = u
    top_rows = []
    top_idx = []
    for _ in range(U):
        big = jnp.max(val)
        idx = jnp.min(jnp.where(val == big, row, S))
        top_idx.append(idx)
        top_rows.append(smat_ref[pl.ds(idx, 1), :])
        val = jnp.where(row == idx, -1e9, val)
    ssub = jnp.concatenate(top_rows, axis=0)
    tmat = jnp.concatenate([jnp.reshape(i_, (1, 1)) for i_ in top_idx], axis=0)

    lane_u = jax.lax.broadcasted_iota(jnp.int32, (U, 1), 0)
    m = jnp.zeros((1, P), jnp.float32)
    blocked = jnp.zeros((U, 1), jnp.bool_)
    for r in range(G_PER_HEAD):
        gains = jnp.sum(jax.nn.relu(ssub - m), axis=-1, keepdims=True)
        gains = jnp.where(blocked, -1e9, gains)
        gbig = jnp.max(gains)
        j = jnp.min(jnp.where(gains == gbig, lane_u, U))
        chosen_abs = jnp.min(jnp.where(lane_u == j, tmat, S))
        o_ref[0, 0, r] = chosen_abs
        blocked = blocked | (lane_u == j)
        picked = jnp.max(jnp.where(lane_u == j, ssub, -jnp.inf),
                         axis=0, keepdims=True)
        m = jnp.maximum(m, picked)


def _pick_globals(q, k, g_eff):
    B, H, S, D = k.shape
    Kmean = k.mean(axis=0)
    Kbar = _normalize_safe(Kmean)
    p = min(PROTO_COUNT, S)
    idxp = np.round(np.linspace(0.0, S - 1, p)).astype(np.int32)
    Qp = _normalize_safe(q.mean(axis=0)[:, idxp, :])
    U = max(g_eff, min(TOP_U, S))
    return pl.pallas_call(
        functools.partial(_select_kernel, S=S, P=p, U=U),
        grid=(H,),
        in_specs=[
            pl.BlockSpec((1, S, D), lambda h: (h, 0, 0)),
            pl.BlockSpec((1, p, D), lambda h: (h, 0, 0)),
        ],
        out_specs=pl.BlockSpec((1, 1, g_eff), lambda h: (h, 0, 0),
                               memory_space=pltpu.SMEM),
        out_shape=jax.ShapeDtypeStruct((H, 1, g_eff), jnp.int32),
        scratch_shapes=[pltpu.VMEM((S, p), jnp.float32)],
        compiler_params=pltpu.CompilerParams(
            dimension_semantics=("arbitrary",)),
    )(Kbar, Qp).reshape(H, g_eff)


def _attn_kernel(g_ref, q_ref, k_ref, v_ref, o_ref, *, S, D, T, L):
    h = pl.program_id(0)
    t = pl.program_id(1)
    t0 = t * T
    base = jnp.clip(t0 - FRAG, 0, S - L)
    scale = 1.0 / np.sqrt(D)

    qb = q_ref[0]
    ks = k_ref[0, pl.ds(base, L), :]
    vs = v_ref[0, pl.ds(base, L), :]

    scores = jax.lax.dot_general(
        qb, ks, (((1,), (1,)), ((), ())),
        preferred_element_type=jnp.float32) * scale

    t_abs = t0 + jax.lax.broadcasted_iota(jnp.int32, (T, L), 0)
    j_abs = base + jax.lax.broadcasted_iota(jnp.int32, (T, L), 1)
    start = jnp.clip(t_abs - FRAG // 2, 0, S - FRAG)
    in_band = (j_abs >= start) & (j_abs < start + FRAG)
    scores = jnp.where(in_band, scores, -1e30)

    rows_k = [k_ref[0, pl.ds(g_ref[h, g], 1), :] for g in range(G_PER_HEAD)]
    rows_v = [v_ref[0, pl.ds(g_ref[h, g], 1), :] for g in range(G_PER_HEAD)]
    pad = jnp.zeros((GPAD - G_PER_HEAD, D), jnp.float32)
    kg = jnp.concatenate(rows_k + [pad], axis=0)
    vg = jnp.concatenate(rows_v + [pad], axis=0)

    gscores = jax.lax.dot_general(
        qb, kg, (((1,), (1,)), ((), ())),
        preferred_element_type=jnp.float32) * scale
    gcol = jax.lax.broadcasted_iota(jnp.int32, (T, GPAD), 1)
    gscores = jnp.where(gcol < G_PER_HEAD, gscores, -1e30)

    m = jnp.maximum(jnp.max(scores, axis=-1, keepdims=True),
                    jnp.max(gscores, axis=-1, keepdims=True))
    pw = jnp.exp(scores - m)
    pg = jnp.exp(gscores - m)
    denom = (jnp.sum(pw, axis=-1, keepdims=True) +
             jnp.sum(pg, axis=-1, keepdims=True))

    out = (jax.lax.dot_general(pw, vs, (((1,), (0,)), ((), ())),
                               preferred_element_type=jnp.float32) +
           jax.lax.dot_general(pg, vg, (((1,), (0,)), ((), ())),
                               preferred_element_type=jnp.float32))
    o_ref[0] = out / denom


def kernel(q, k, v):
    B, H, S, D = q.shape
    g_idx = _pick_globals(q, k, G_PER_HEAD)

    qh = q.reshape(H, S, D)
    kh = k.reshape(H, S, D)
    vh = v.reshape(H, S, D)

    T = T_Q
    L = T + 2 * FRAG

    grid_spec = pltpu.PrefetchScalarGridSpec(
        num_scalar_prefetch=1,
        grid=(H, S // T),
        in_specs=[
            pl.BlockSpec((1, T, D), lambda h, t, g: (h, t, 0)),
            pl.BlockSpec((1, S, D), lambda h, t, g: (h, 0, 0)),
            pl.BlockSpec((1, S, D), lambda h, t, g: (h, 0, 0)),
        ],
        out_specs=pl.BlockSpec((1, T, D), lambda h, t, g: (h, t, 0)),
    )

    out = pl.pallas_call(
        functools.partial(_attn_kernel, S=S, D=D, T=T, L=L),
        grid_spec=grid_spec,
        out_shape=jax.ShapeDtypeStruct((H, S, D), jnp.float32),
        compiler_params=pltpu.CompilerParams(
            dimension_semantics=("arbitrary", "arbitrary")),
    )(g_idx, qh, kh, vh)

    return out.reshape(B, H, S, D)


# speedup vs baseline: 1.1299x; 1.1299x over previous
import functools

import jax
import jax.numpy as jnp
import numpy as np
from jax.experimental import pallas as pl
from jax.experimental.pallas import tpu as pltpu

FRAG = 32
G_PER_HEAD = 3
PROTO_COUNT = 16
TOP_U = 8
TOPK_FRAC = 0.2
W_MEAN, W_MAX, W_TOPK, W_STD = 1.0, 0.6, 0.4, 0.2

T_Q = 256
GPAD = 8


def _normalize_safe(x, eps=1e-6):
    n = jnp.linalg.norm(x, axis=-1, keepdims=True)
    return x / jnp.maximum(n, eps)


def _select_kernel(kbar_ref, qp_ref, o_ref, *, S, P, U):
    kb = kbar_ref[0]
    qp = qp_ref[0]
    smt = jax.nn.relu(jax.lax.dot_general(
        qp, kb, (((1,), (1,)), ((), ())),
        preferred_element_type=jnp.float32))

    sub = jax.lax.broadcasted_iota(jnp.int32, (P, S), 0)
    mean = jnp.mean(smt, axis=0, keepdims=True)
    mx = jnp.max(smt, axis=0, keepdims=True)
    kq = max(1, int(round(P * TOPK_FRAC)))
    cur = smt
    s3 = jnp.zeros((1, S), jnp.float32)
    for _ in range(kq):
        mi = jnp.max(cur, axis=0, keepdims=True)
        first = jnp.min(jnp.where(cur == mi, sub, P), axis=0, keepdims=True)
        s3 = s3 + mi
        cur = jnp.where(sub == first, -jnp.inf, cur)
    topk_mean = s3 / float(kq)
    dev = smt - mean
    std = jnp.sqrt(jnp.sum(dev * dev, axis=0, keepdims=True) / (P - 1))
    u = W_MEAN * mean + W_MAX * mx + W_TOPK * topk_mean + W_STD * std

    col = jax.lax.broadcasted_iota(jnp.int32, (1, S), 1)
    col_ps = jax.lax.broadcasted_iota(jnp.int32, (P, S), 1)
    val = u
    top_cols = []
    top_idx = []
    for _ in range(U):
        big = jnp.max(val)
        idx = jnp.min(jnp.where(val == big, col, S))
        top_idx.append(idx)
        top_cols.append(jnp.max(jnp.where(col_ps == idx, smt, -jnp.inf),
                                axis=1, keepdims=True))
        val = jnp.where(col == idx, -1e9, val)
    ssub = jnp.concatenate(top_cols, axis=1)
    tvec = jnp.concatenate([jnp.reshape(i_, (1, 1)) for i_ in top_idx],
                           axis=1)

    col_u = jax.lax.broadcasted_iota(jnp.int32, (1, U), 1)
    col_pu = jax.lax.broadcasted_iota(jnp.int32, (P, U), 1)
    m = jnp.zeros((P, 1), jnp.float32)
    blocked = jnp.zeros((1, U), jnp.bool_)
    for r in range(G_PER_HEAD):
        gains = jnp.sum(jax.nn.relu(ssub - m), axis=0, keepdims=True)
        gains = jnp.where(blocked, -1e9, gains)
        gbig = jnp.max(gains)
        j = jnp.min(jnp.where(gains == gbig, col_u, U))
        o_ref[0, 0, r] = jnp.min(jnp.where(col_u == j, tvec, S))
        blocked = blocked | (col_u == j)
        picked = jnp.max(jnp.where(col_pu == j, ssub, -jnp.inf),
                         axis=1, keepdims=True)
        m = jnp.maximum(m, picked)


def _pick_globals(q, k, g_eff):
    B, H, S, D = k.shape
    Kmean = k.mean(axis=0)
    Kbar = _normalize_safe(Kmean)
    p = min(PROTO_COUNT, S)
    idxp = np.round(np.linspace(0.0, S - 1, p)).astype(np.int32)
    Qp = _normalize_safe(q.mean(axis=0)[:, idxp, :])
    U = max(g_eff, min(TOP_U, S))
    return pl.pallas_call(
        functools.partial(_select_kernel, S=S, P=p, U=U),
        grid=(H,),
        in_specs=[
            pl.BlockSpec((1, S, D), lambda h: (h, 0, 0)),
            pl.BlockSpec((1, p, D), lambda h: (h, 0, 0)),
        ],
        out_specs=pl.BlockSpec((1, 1, g_eff), lambda h: (h, 0, 0),
                               memory_space=pltpu.SMEM),
        out_shape=jax.ShapeDtypeStruct((H, 1, g_eff), jnp.int32),
        compiler_params=pltpu.CompilerParams(
            dimension_semantics=("parallel",)),
    )(Kbar, Qp).reshape(H, g_eff)


def _attn_kernel(g_ref, q_ref, k_ref, v_ref, o_ref, *, S, D, T, L):
    h = pl.program_id(0)
    t = pl.program_id(1)
    t0 = t * T
    base = jnp.clip(t0 - FRAG // 2, 0, S - L)
    scale = 1.0 / np.sqrt(D)

    qb = q_ref[0, 0]
    ks = k_ref[0, 0, pl.ds(base, L), :]
    vs = v_ref[0, 0, pl.ds(base, L), :]

    scores = jax.lax.dot_general(
        qb, ks, (((1,), (1,)), ((), ())),
        preferred_element_type=jnp.float32) * scale

    t_abs = t0 + jax.lax.broadcasted_iota(jnp.int32, (T, L), 0)
    j_abs = base + jax.lax.broadcasted_iota(jnp.int32, (T, L), 1)
    start = jnp.clip(t_abs - FRAG // 2, 0, S - FRAG)
    in_band = (j_abs >= start) & (j_abs < start + FRAG)
    scores = jnp.where(in_band, scores, -1e30)

    rows_k = [k_ref[0, 0, pl.ds(g_ref[h, g], 1), :] for g in range(G_PER_HEAD)]
    rows_v = [v_ref[0, 0, pl.ds(g_ref[h, g], 1), :] for g in range(G_PER_HEAD)]
    pad = jnp.zeros((GPAD - G_PER_HEAD, D), jnp.float32)
    kg = jnp.concatenate(rows_k + [pad], axis=0)
    vg = jnp.concatenate(rows_v + [pad], axis=0)

    gscores = jax.lax.dot_general(
        qb, kg, (((1,), (1,)), ((), ())),
        preferred_element_type=jnp.float32) * scale
    gcol = jax.lax.broadcasted_iota(jnp.int32, (T, GPAD), 1)
    gscores = jnp.where(gcol < G_PER_HEAD, gscores, -1e30)

    m = jnp.maximum(jnp.max(scores, axis=-1, keepdims=True),
                    jnp.max(gscores, axis=-1, keepdims=True))
    pw = jnp.exp(scores - m)
    pg = jnp.exp(gscores - m)
    denom = (jnp.sum(pw, axis=-1, keepdims=True) +
             jnp.sum(pg, axis=-1, keepdims=True))

    out = (jax.lax.dot_general(pw, vs, (((1,), (0,)), ((), ())),
                               preferred_element_type=jnp.float32) +
           jax.lax.dot_general(pg, vg, (((1,), (0,)), ((), ())),
                               preferred_element_type=jnp.float32))
    o_ref[0, 0] = out / denom


def kernel(q, k, v):
    B, H, S, D = q.shape
    g_idx = _pick_globals(q, k, G_PER_HEAD)

    T = T_Q
    L = T + FRAG

    grid_spec = pltpu.PrefetchScalarGridSpec(
        num_scalar_prefetch=1,
        grid=(H, S // T),
        in_specs=[
            pl.BlockSpec((1, 1, T, D), lambda h, t, g: (0, h, t, 0)),
            pl.BlockSpec((1, 1, S, D), lambda h, t, g: (0, h, 0, 0)),
            pl.BlockSpec((1, 1, S, D), lambda h, t, g: (0, h, 0, 0)),
        ],
        out_specs=pl.BlockSpec((1, 1, T, D), lambda h, t, g: (0, h, t, 0)),
    )

    out = pl.pallas_call(
        functools.partial(_attn_kernel, S=S, D=D, T=T, L=L),
        grid_spec=grid_spec,
        out_shape=jax.ShapeDtypeStruct((B, H, S, D), jnp.float32),
        compiler_params=pltpu.CompilerParams(
            dimension_semantics=("parallel", "arbitrary")),
    )(g_idx, q, k, v)

    return out


# speedup vs baseline: 1.3931x; 1.2330x over previous
import functools

import jax
import jax.numpy as jnp
import numpy as np
from jax.experimental import pallas as pl
from jax.experimental.pallas import tpu as pltpu

FRAG = 32
G_PER_HEAD = 3
PROTO_COUNT = 16
TOP_U = 8
TOPK_FRAC = 0.2
W_MEAN, W_MAX, W_TOPK, W_STD = 1.0, 0.6, 0.4, 0.2

T_Q = 512
GPAD = 8


def _normalize_safe(x, eps=1e-6):
    n = jnp.linalg.norm(x, axis=-1, keepdims=True)
    return x / jnp.maximum(n, eps)


def _fused_kernel(qp_ref, q_ref, k_ref, v_ref, o_ref, g_s, *, S, D, T, L, P, U):
    h = pl.program_id(0)
    t = pl.program_id(1)

    @pl.when(t == 0)
    def _select():
        kh = k_ref[0, 0]
        nrm = jnp.sqrt(jnp.sum(kh * kh, axis=-1, keepdims=True))
        kb = kh / jnp.maximum(nrm, 1e-6)
        qp = qp_ref[0]
        smt = jax.nn.relu(jax.lax.dot_general(
            qp, kb, (((1,), (1,)), ((), ())),
            preferred_element_type=jnp.float32))

        sub = jax.lax.broadcasted_iota(jnp.int32, (P, S), 0)
        mean = jnp.mean(smt, axis=0, keepdims=True)
        mx = jnp.max(smt, axis=0, keepdims=True)
        kq = max(1, int(round(P * TOPK_FRAC)))
        cur = smt
        s3 = jnp.zeros((1, S), jnp.float32)
        for _ in range(kq):
            mi = jnp.max(cur, axis=0, keepdims=True)
            first = jnp.min(jnp.where(cur == mi, sub, P), axis=0,
                            keepdims=True)
            s3 = s3 + mi
            cur = jnp.where(sub == first, -jnp.inf, cur)
        topk_mean = s3 / float(kq)
        dev = smt - mean
        std = jnp.sqrt(jnp.sum(dev * dev, axis=0, keepdims=True) / (P - 1))
        u = (W_MEAN * mean + W_MAX * mx + W_TOPK * topk_mean
             + W_STD * std)

        col = jax.lax.broadcasted_iota(jnp.int32, (1, S), 1)
        col_ps = jax.lax.broadcasted_iota(jnp.int32, (P, S), 1)
        val = u
        top_cols = []
        top_idx = []
        for _ in range(U):
            big = jnp.max(val, axis=1, keepdims=True)
            idxv = jnp.min(jnp.where(val == big, col, S), axis=1,
                           keepdims=True)
            top_idx.append(idxv)
            top_cols.append(jnp.max(jnp.where(col_ps == idxv, smt, -jnp.inf),
                                    axis=1, keepdims=True))
            val = jnp.where(col == idxv, -1e9, val)
        ssub = jnp.concatenate(top_cols, axis=1)
        tvec = jnp.concatenate(top_idx, axis=1)

        col_u = jax.lax.broadcasted_iota(jnp.int32, (1, U), 1)
        col_pu = jax.lax.broadcasted_iota(jnp.int32, (P, U), 1)
        m = jnp.zeros((P, 1), jnp.float32)
        blocked = jnp.zeros((1, U), jnp.bool_)
        for r in range(G_PER_HEAD):
            gains = jnp.sum(jax.nn.relu(ssub - m), axis=0, keepdims=True)
            gains = jnp.where(blocked, -1e9, gains)
            gbig = jnp.max(gains, axis=1, keepdims=True)
            j = jnp.min(jnp.where(gains == gbig, col_u, U), axis=1,
                        keepdims=True)
            g_s[0, r] = jnp.min(jnp.where(col_u == j, tvec, S))
            blocked = blocked | (col_u == j)
            picked = jnp.max(jnp.where(col_pu == j, ssub, -jnp.inf),
                             axis=1, keepdims=True)
            m = jnp.maximum(m, picked)

    t0 = t * T
    base = jnp.clip(t0 - FRAG // 2, 0, S - L)
    scale = 1.0 / np.sqrt(D)

    qb = q_ref[0, 0] * scale
    ks = k_ref[0, 0, pl.ds(base, L), :]
    vs = v_ref[0, 0, pl.ds(base, L), :]

    scores = jax.lax.dot_general(
        qb, ks, (((1,), (1,)), ((), ())),
        preferred_element_type=jnp.float32)

    t_abs = t0 + jax.lax.broadcasted_iota(jnp.int32, (T, L), 0)
    j_abs = base + jax.lax.broadcasted_iota(jnp.int32, (T, L), 1)
    start = jnp.clip(t_abs - FRAG // 2, 0, S - FRAG)
    in_band = (j_abs >= start) & (j_abs < start + FRAG)
    scores = jnp.where(in_band, scores, -1e30)

    rows_k = [k_ref[0, 0, pl.ds(g_s[0, g], 1), :] for g in range(G_PER_HEAD)]
    rows_v = [v_ref[0, 0, pl.ds(g_s[0, g], 1), :] for g in range(G_PER_HEAD)]
    pad = jnp.zeros((GPAD - G_PER_HEAD, D), jnp.float32)
    kg = jnp.concatenate(rows_k + [pad], axis=0)
    vg = jnp.concatenate(rows_v + [pad], axis=0)

    gscores = jax.lax.dot_general(
        qb, kg, (((1,), (1,)), ((), ())),
        preferred_element_type=jnp.float32)
    gcol = jax.lax.broadcasted_iota(jnp.int32, (T, GPAD), 1)
    gscores = jnp.where(gcol < G_PER_HEAD, gscores, -1e30)

    mrow = jnp.maximum(jnp.max(scores, axis=-1, keepdims=True),
                       jnp.max(gscores, axis=-1, keepdims=True))
    pw = jnp.exp(scores - mrow)
    pg = jnp.exp(gscores - mrow)
    denom = (jnp.sum(pw, axis=-1, keepdims=True) +
             jnp.sum(pg, axis=-1, keepdims=True))

    out = (jax.lax.dot_general(pw, vs, (((1,), (0,)), ((), ())),
                               preferred_element_type=jnp.float32) +
           jax.lax.dot_general(pg, vg, (((1,), (0,)), ((), ())),
                               preferred_element_type=jnp.float32))
    o_ref[0, 0] = out / denom


def kernel(q, k, v):
    B, H, S, D = q.shape
    P = min(PROTO_COUNT, S)
    U = max(G_PER_HEAD, min(TOP_U, S))
    idxp = np.round(np.linspace(0.0, S - 1, P)).astype(np.int32)
    Qp = _normalize_safe(q.mean(axis=0)[:, idxp, :])

    T = T_Q
    L = T + FRAG

    out = pl.pallas_call(
        functools.partial(_fused_kernel, S=S, D=D, T=T, L=L, P=P, U=U),
        grid=(H, S // T),
        in_specs=[
            pl.BlockSpec((1, P, D), lambda h, t: (h, 0, 0)),
            pl.BlockSpec((1, 1, T, D), lambda h, t: (0, h, t, 0)),
            pl.BlockSpec((1, 1, S, D), lambda h, t: (0, h, 0, 0)),
            pl.BlockSpec((1, 1, S, D), lambda h, t: (0, h, 0, 0)),
        ],
        out_specs=pl.BlockSpec((1, 1, T, D), lambda h, t: (0, h, t, 0)),
        out_shape=jax.ShapeDtypeStruct((B, H, S, D), jnp.float32),
        scratch_shapes=[pltpu.SMEM((1, GPAD), jnp.int32)],
        compiler_params=pltpu.CompilerParams(
            dimension_semantics=("parallel", "arbitrary")),
    )(Qp, q, k, v)

    return out
